# baseline (speedup 1.0000x reference)
import functools

import jax
import jax.numpy as jnp
from jax import lax
from jax.experimental import pallas as pl
from jax.experimental.pallas import tpu as pltpu
from jax.experimental.pallas import tpu_sc as plsc

VOCAB = 1_000_000
EMB = 64
PADE = 128
NBUF = 2

_info = plsc.get_sparse_core_info()
NC, NS = _info.num_cores, _info.num_subcores
NW = NC * NS


def _build(n_seq: int, seq_len: int):
    seq_per_w = n_seq // NW
    c0 = (seq_len // PADE) * PADE
    c1 = seq_len - c0

    mesh = plsc.VectorSubcoreMesh(core_axis_name="c", subcore_axis_name="s")

    @functools.partial(
        pl.kernel,
        out_type=jax.ShapeDtypeStruct((n_seq, seq_len, EMB), jnp.float32),
        mesh=mesh,
        scratch_types=[
            pltpu.VMEM((seq_per_w, seq_len), jnp.int32),
            pltpu.VMEM((NBUF, seq_len, PADE), jnp.float32),
            [pltpu.SemaphoreType.DMA] * NBUF,
            [pltpu.SemaphoreType.DMA] * NBUF,
        ],
        compiler_params=pltpu.CompilerParams(use_tc_tiling_on_sc=False),
    )
    def emb(tok_hbm, table_hbm, out_hbm, idx_v, rows_v, gsems, wsems):
        wid = lax.axis_index("s") * NC + lax.axis_index("c")
        seq0 = wid * seq_per_w

        pltpu.sync_copy(tok_hbm.at[pl.ds(seq0, seq_per_w)], idx_v)

        def fire_gather(s_local, b):
            pltpu.async_copy(
                table_hbm.at[idx_v.at[s_local, pl.ds(0, c0)]],
                rows_v.at[b, pl.ds(0, c0)], gsems[b])
            pltpu.async_copy(
                table_hbm.at[idx_v.at[s_local, pl.ds(c0, c1)]],
                rows_v.at[b, pl.ds(c0, c1)], gsems[b])

        def wait_gather(s_local, b):
            pltpu.make_async_copy(
                table_hbm.at[idx_v.at[s_local, pl.ds(0, c0)]],
                rows_v.at[b, pl.ds(0, c0)], gsems[b]).wait()
            pltpu.make_async_copy(
                table_hbm.at[idx_v.at[s_local, pl.ds(c0, c1)]],
                rows_v.at[b, pl.ds(c0, c1)], gsems[b]).wait()

        def fire_wb(s_local, b):
            pltpu.async_copy(
                rows_v.at[b, :, pl.ds(0, EMB)],
                out_hbm.at[seq0 + s_local], wsems[b])

        def wait_wb(s_local, b):
            pltpu.make_async_copy(
                rows_v.at[b, :, pl.ds(0, EMB)],
                out_hbm.at[seq0 + s_local], wsems[b]).wait()

        for b in range(NBUF):
            fire_gather(b, b)

        def step(t, carry):
            for b in range(NBUF):
                s = NBUF + t * NBUF + b
                wait_gather(s - NBUF, b)
                fire_wb(s - NBUF, b)
                wait_wb(s - NBUF, b)
                fire_gather(s, b)
            return carry

        lax.fori_loop(0, (seq_per_w - NBUF) // NBUF, step, 0)

        for b in range(NBUF):
            s = seq_per_w - NBUF + b
            wait_gather(s, b)
            fire_wb(s, b)
        for b in range(NBUF):
            s = seq_per_w - NBUF + b
            wait_wb(s, b)

    return emb


def kernel(token, table):
    n_seq, seq_len = token.shape
    tablep = jnp.pad(table, ((0, 0), (0, PADE - EMB)))
    return _build(n_seq, seq_len)(token.astype(jnp.int32), tablep)


# speedup vs baseline: 1.3423x; 1.3423x over previous
import functools

import jax
import jax.numpy as jnp
from jax import lax
from jax.experimental import pallas as pl
from jax.experimental.pallas import tpu as pltpu
from jax.experimental.pallas import tpu_sc as plsc

VOCAB = 1_000_000
EMB = 64
PADE = 128
NBUF = 2

_info = plsc.get_sparse_core_info()
NC, NS = _info.num_cores, _info.num_subcores
NW = NC * NS


def _build(n_seq: int, seq_len: int):
    seq_per_w = n_seq // NW
    c0 = (seq_len // 128) * 128
    c1 = seq_len - c0

    mesh = plsc.VectorSubcoreMesh(core_axis_name="c", subcore_axis_name="s")

    @functools.partial(
        pl.kernel,
        out_type=jax.ShapeDtypeStruct((n_seq, seq_len, PADE), jnp.float32),
        mesh=mesh,
        scratch_types=[
            pltpu.VMEM((seq_per_w, seq_len), jnp.int32),
            pltpu.VMEM((NBUF, seq_len, EMB), jnp.float32),
            [pltpu.SemaphoreType.DMA] * NBUF,
            [pltpu.SemaphoreType.DMA] * NBUF,
        ],
        compiler_params=pltpu.CompilerParams(use_tc_tiling_on_sc=False),
    )
    def emb(tok_hbm, table_hbm, out_hbm, idx_v, rows_v, gsems, wsems):
        wid = lax.axis_index("s") * NC + lax.axis_index("c")
        seq0 = wid * seq_per_w

        pltpu.sync_copy(tok_hbm.at[pl.ds(seq0, seq_per_w)], idx_v)

        def fire_gather(s_local, b):
            pltpu.async_copy(
                table_hbm.at[idx_v.at[s_local, pl.ds(0, c0)]],
                rows_v.at[b, pl.ds(0, c0)], gsems[b])
            pltpu.async_copy(
                table_hbm.at[idx_v.at[s_local, pl.ds(c0, c1)]],
                rows_v.at[b, pl.ds(c0, c1)], gsems[b])

        def wait_gather(s_local, b):
            pltpu.make_async_copy(
                table_hbm.at[idx_v.at[s_local, pl.ds(0, c0)]],
                rows_v.at[b, pl.ds(0, c0)], gsems[b]).wait()
            pltpu.make_async_copy(
                table_hbm.at[idx_v.at[s_local, pl.ds(c0, c1)]],
                rows_v.at[b, pl.ds(c0, c1)], gsems[b]).wait()

        def fire_wb(s_local, b):
            pltpu.async_copy(
                rows_v.at[b],
                out_hbm.at[seq0 + s_local, :, pl.ds(0, EMB)], wsems[b])

        def wait_wb(s_local, b):
            pltpu.make_async_copy(
                rows_v.at[b],
                out_hbm.at[seq0 + s_local, :, pl.ds(0, EMB)], wsems[b]).wait()

        for b in range(NBUF):
            fire_gather(b, b)

        def step(t, carry):
            for b in range(NBUF):
                s = NBUF + t * NBUF + b
                wait_gather(s - NBUF, b)
                fire_wb(s - NBUF, b)
                wait_wb(s - NBUF, b)
                fire_gather(s, b)
            return carry

        lax.fori_loop(0, (seq_per_w - NBUF) // NBUF, step, 0)

        for b in range(NBUF):
            s = seq_per_w - NBUF + b
            wait_gather(s, b)
            fire_wb(s, b)
        for b in range(NBUF):
            s = seq_per_w - NBUF + b
            wait_wb(s, b)

    return emb


def kernel(token, table):
    n_seq, seq_len = token.shape
    out = _build(n_seq, seq_len)(token.astype(jnp.int32), table)
    return out[:, :, :EMB]
